# baseline (speedup 1.0000x reference)
import dataclasses
import functools

import jax
import jax.numpy as jnp
from jax import lax
from jax.experimental import pallas as pl
from jax.experimental.pallas import tpu as pltpu
from jax.experimental.pallas import tpu_sc as plsc

N = 10000
E = 320000
D = 128
T = 4
NC = 2
NS = 16
NW = NC * NS
EPT = E // NW
CH = 80
NCHUNK = EPT // CH
RB = 80
NRC = N // RB
ZROUNDS = (NRC + NS - 1) // NS


NSLOT = 4
MAIN = NCHUNK - 1
NROUND = MAIN // NSLOT
REC = 256
ZB = 40
NZC = N // ZB
ZZROUNDS = (NZC + NS - 1) // NS


def _sc_pass(x_flat, packed, g_count):
    mesh = plsc.VectorSubcoreMesh(core_axis_name="c", subcore_axis_name="s")
    cp = pltpu.CompilerParams()
    if "needs_layout_passes" in pltpu.CompilerParams.__dataclass_fields__:
        cp = dataclasses.replace(cp, needs_layout_passes=False)

    @functools.partial(
        pl.kernel,
        mesh=mesh,
        compiler_params=cp,
        out_type=jax.ShapeDtypeStruct((NC * g_count * N, D), jnp.float32),
        scratch_types=(
            [pltpu.VMEM((REC,), jnp.int32)] * NSLOT
            + [pltpu.VMEM((CH,), jnp.int32)] * NSLOT
            + [pltpu.VMEM((CH,), jnp.int32)] * NSLOT
            + [pltpu.VMEM((CH, D), jnp.float32)] * NSLOT
            + [pltpu.VMEM((CH,), jnp.float32)] * NSLOT
            + [
                pltpu.VMEM((ZB, D), jnp.float32),
                pltpu.VMEM_SHARED((N, D), jnp.float32),
            ]
            + [pltpu.SemaphoreType.DMA] * (3 * NSLOT)
        ),
    )
    def k(x_hbm, pk_hbm, out_hbm, *scratch):
        idx_v = scratch[0:NSLOT]
        srcg = scratch[NSLOT:2 * NSLOT]
        dstb = scratch[2 * NSLOT:3 * NSLOT]
        rows_v = scratch[3 * NSLOT:4 * NSLOT]
        wbuf = scratch[4 * NSLOT:5 * NSLOT]
        zbuf_v = scratch[5 * NSLOT]
        acc = scratch[5 * NSLOT + 1]
        isem = scratch[5 * NSLOT + 2:5 * NSLOT + 2 + NSLOT]
        gsem = scratch[5 * NSLOT + 2 + NSLOT:5 * NSLOT + 2 + 2 * NSLOT]
        ssem = scratch[5 * NSLOT + 2 + 2 * NSLOT:]

        cid = lax.axis_index("c")
        sid = lax.axis_index("s")
        wid = cid * NS + sid
        cbase = wid * NCHUNK

        def idx_copy(c, s):
            off = (cbase + c) * REC
            return pltpu.make_async_copy(
                pk_hbm.at[pl.ds(off, REC)], idx_v[s], isem[s])

        def gather_copy(s):
            return pltpu.make_async_copy(
                x_hbm.at[srcg[s]], rows_v[s], gsem[s])

        def scatter_copy(s):
            return pltpu.make_async_copy(
                rows_v[s], acc.at[dstb[s]], ssem[s])

        def s1_prep(g, s):
            goff = jnp.full((16,), g * N, jnp.int32)
            for t in range(CH // 16):
                sl = pl.ds(t * 16, 16)
                srcg[s][sl] = idx_v[s][pl.ds(t * 16, 16)] + goff
                dstb[s][sl] = idx_v[s][pl.ds(CH + t * 16, 16)]
                wbuf[s][sl] = plsc.bitcast(
                    idx_v[s][pl.ds(2 * CH + t * 16, 16)], jnp.float32)
            gather_copy(s).start()

        def s2_scale(s):
            @plsc.parallel_loop(0, CH, 1, unroll=8)
            def _(j):
                wv = plsc.load_gather(
                    wbuf[s], [jnp.full((16,), j, jnp.int32)])
                for f in range(D // 16):
                    sl = (j, pl.ds(f * 16, 16))
                    rows_v[s][sl] = rows_v[s][sl] * wv

        @pl.loop(0, ZB)
        def _(r):
            for f in range(D // 16):
                zbuf_v[r, pl.ds(f * 16, 16)] = jnp.zeros((16,), jnp.float32)

        @pl.loop(0, g_count)
        def _(g):
            @pl.loop(0, ZZROUNDS)
            def _(r):
                c = sid + r * NS

                @pl.when(c < NZC)
                def _():
                    pltpu.sync_copy(zbuf_v, acc.at[pl.ds(c * ZB, ZB), :])

            plsc.subcore_barrier()

            idx_copy(0, 0).start()
            idx_copy(1, 1).start()
            idx_copy(2, 2).start()
            idx_copy(0, 0).wait()
            s1_prep(g, 0)
            idx_copy(1, 1).wait()
            s1_prep(g, 1)

            @pl.loop(0, NROUND)
            def _(t):
                c0 = t * NSLOT
                for p in range(NSLOT):
                    c = c0 + p
                    s = p
                    s2 = (p + 2) % NSLOT
                    s3 = (p + 3) % NSLOT

                    @pl.when(c + 3 < NCHUNK)
                    def _():
                        idx_copy(c + 3, s3).start()

                    @pl.when(c + 2 < NCHUNK)
                    def _():
                        idx_copy(c + 2, s2).wait()

                        @pl.when(c + 2 >= NSLOT)
                        def _():
                            scatter_copy(s2).wait()

                        s1_prep(g, s2)

                    gather_copy(s).wait()
                    s2_scale(s)
                    scatter_copy(s).start(add=True)

            gather_copy(0).wait()
            s2_scale(0)
            scatter_copy(0).start(add=True)

            for s in range(NSLOT):
                scatter_copy(s).wait()
            plsc.subcore_barrier()

            obase = (cid * g_count + g) * N

            @pl.loop(0, ZROUNDS)
            def _(r):
                c = sid + r * NS

                @pl.when(c < NRC)
                def _():
                    pltpu.sync_copy(
                        acc.at[pl.ds(c * RB, RB), :],
                        out_hbm.at[pl.ds(obase + c * RB, RB), :])

            plsc.subcore_barrier()

    return k(x_flat, packed)


BN = 400


def _tc_stage(partial, W, mask, relu):
    per_sample = partial.shape[1] == T

    def body(p_ref, w_ref, m_ref, o_ref):
        p = p_ref[0, 0] + p_ref[1, 0]
        pw = jnp.dot(p, w_ref[...] * m_ref[0],
                     preferred_element_type=jnp.float32)
        if relu:
            pw = jnp.maximum(pw, 0.0)
        o_ref[0] = pw

    if per_sample:
        p_map = lambda n, i: (0, i, n, 0)
    else:
        p_map = lambda n, i: (0, 0, n, 0)
    return pl.pallas_call(
        body,
        grid=(N // BN, T),
        in_specs=[
            pl.BlockSpec((2, 1, BN, D), p_map),
            pl.BlockSpec((D, D), lambda n, i: (0, 0)),
            pl.BlockSpec((1, D, D), lambda n, i: (i, 0, 0)),
        ],
        out_specs=pl.BlockSpec((1, BN, D), lambda n, i: (i, n, 0)),
        out_shape=jax.ShapeDtypeStruct((T, N, D), jnp.float32),
    )(partial, W, mask)


def kernel(feature, edge_weight, W1, W2, mask1, mask2, edge_index):
    w_bits = jax.lax.bitcast_convert_type(edge_weight, jnp.int32)
    packed = jnp.concatenate(
        [edge_index, w_bits[None]], axis=0)
    packed = packed.reshape(3, NW * NCHUNK, CH).transpose(1, 0, 2)
    packed = jnp.pad(packed.reshape(NW * NCHUNK, 3 * CH),
                     ((0, 0), (0, REC - 3 * CH))).reshape(-1)

    p1 = _sc_pass(feature, packed, 1)
    p1 = p1.reshape(NC, 1, N, D)

    H = _tc_stage(p1, W1, mask1, relu=True)

    p2 = _sc_pass(H.reshape(T * N, D), packed, T)
    p2 = p2.reshape(NC, T, N, D)

    return _tc_stage(p2, W2, mask2, relu=False)


# speedup vs baseline: 1.1377x; 1.1377x over previous
import dataclasses
import functools

import jax
import jax.numpy as jnp
from jax import lax
from jax.experimental import pallas as pl
from jax.experimental.pallas import tpu as pltpu
from jax.experimental.pallas import tpu_sc as plsc

N = 10000
E = 320000
D = 128
T = 4
NC = 2
NS = 16
NW = NC * NS
EPT = E // NW
CH = 80
NCHUNK = EPT // CH
RB = 80
NRC = N // RB
ZROUNDS = (NRC + NS - 1) // NS


NSLOT = 4
MAIN = NCHUNK - 1
NROUND = MAIN // NSLOT
REC = 256
ZB = 40
NZC = N // ZB
ZZROUNDS = (NZC + NS - 1) // NS


def _sc_pass(x_flat, packed, g_count):
    mesh = plsc.VectorSubcoreMesh(core_axis_name="c", subcore_axis_name="s")
    cp = pltpu.CompilerParams()
    if "needs_layout_passes" in pltpu.CompilerParams.__dataclass_fields__:
        cp = dataclasses.replace(cp, needs_layout_passes=False)

    @functools.partial(
        pl.kernel,
        mesh=mesh,
        compiler_params=cp,
        out_type=jax.ShapeDtypeStruct((NC * g_count * N, D), jnp.float32),
        scratch_types=(
            [pltpu.VMEM((REC,), jnp.int32)] * NSLOT
            + [pltpu.VMEM((CH,), jnp.int32)] * NSLOT
            + [pltpu.VMEM((CH,), jnp.int32)] * NSLOT
            + [pltpu.VMEM((CH, D), jnp.float32)] * NSLOT
            + [pltpu.VMEM((CH,), jnp.float32)] * NSLOT
            + [
                pltpu.VMEM((ZB, D), jnp.float32),
                pltpu.VMEM_SHARED((N, D), jnp.float32),
            ]
            + [pltpu.SemaphoreType.DMA] * (3 * NSLOT)
        ),
    )
    def k(x_hbm, pk_hbm, out_hbm, *scratch):
        idx_v = scratch[0:NSLOT]
        srcg = scratch[NSLOT:2 * NSLOT]
        dstb = scratch[2 * NSLOT:3 * NSLOT]
        rows_v = scratch[3 * NSLOT:4 * NSLOT]
        wbuf = scratch[4 * NSLOT:5 * NSLOT]
        zbuf_v = scratch[5 * NSLOT]
        acc = scratch[5 * NSLOT + 1]
        isem = scratch[5 * NSLOT + 2:5 * NSLOT + 2 + NSLOT]
        gsem = scratch[5 * NSLOT + 2 + NSLOT:5 * NSLOT + 2 + 2 * NSLOT]
        ssem = scratch[5 * NSLOT + 2 + 2 * NSLOT:]

        cid = lax.axis_index("c")
        sid = lax.axis_index("s")
        wid = cid * NS + sid
        cbase = wid * NCHUNK

        def idx_copy(c, s):
            off = (cbase + c) * REC
            return pltpu.make_async_copy(
                pk_hbm.at[pl.ds(off, REC)], idx_v[s], isem[s])

        def gather_copy(s):
            return pltpu.make_async_copy(
                x_hbm.at[srcg[s]], rows_v[s], gsem[s])

        def scatter_copy(s):
            return pltpu.make_async_copy(
                rows_v[s], acc.at[dstb[s]], ssem[s])

        def s1_prep(g, s):
            goff = jnp.full((16,), g * N, jnp.int32)
            for t in range(CH // 16):
                sl = pl.ds(t * 16, 16)
                srcg[s][sl] = idx_v[s][pl.ds(t * 16, 16)] + goff
                dstb[s][sl] = idx_v[s][pl.ds(CH + t * 16, 16)]
                wbuf[s][sl] = plsc.bitcast(
                    idx_v[s][pl.ds(2 * CH + t * 16, 16)], jnp.float32)
            gather_copy(s).start()

        def s2_scale(s):
            @plsc.parallel_loop(0, CH, 1, unroll=4)
            def _(j):
                wv = plsc.load_gather(
                    wbuf[s], [jnp.full((16,), j, jnp.int32)])
                for f in range(D // 16):
                    sl = (j, pl.ds(f * 16, 16))
                    rows_v[s][sl] = rows_v[s][sl] * wv

        @pl.loop(0, ZB)
        def _(r):
            for f in range(D // 16):
                zbuf_v[r, pl.ds(f * 16, 16)] = jnp.zeros((16,), jnp.float32)

        @pl.loop(0, g_count)
        def _(g):
            @pl.loop(0, ZZROUNDS)
            def _(r):
                c = sid + r * NS

                @pl.when(c < NZC)
                def _():
                    pltpu.sync_copy(zbuf_v, acc.at[pl.ds(c * ZB, ZB), :])

            plsc.subcore_barrier()

            idx_copy(0, 0).start()
            idx_copy(1, 1).start()
            idx_copy(2, 2).start()
            idx_copy(0, 0).wait()
            s1_prep(g, 0)
            idx_copy(1, 1).wait()
            s1_prep(g, 1)

            @pl.loop(0, NROUND)
            def _(t):
                c0 = t * NSLOT
                for p in range(NSLOT):
                    c = c0 + p
                    s = p
                    s2 = (p + 2) % NSLOT
                    s3 = (p + 3) % NSLOT

                    @pl.when(c + 3 < NCHUNK)
                    def _():
                        idx_copy(c + 3, s3).start()

                    @pl.when(c + 2 < NCHUNK)
                    def _():
                        idx_copy(c + 2, s2).wait()

                        @pl.when(c + 2 >= NSLOT)
                        def _():
                            scatter_copy(s2).wait()

                        s1_prep(g, s2)

                    gather_copy(s).wait()
                    s2_scale(s)
                    scatter_copy(s).start(add=True)

            gather_copy(0).wait()
            s2_scale(0)
            scatter_copy(0).start(add=True)

            for s in range(NSLOT):
                scatter_copy(s).wait()
            plsc.subcore_barrier()

            obase = (cid * g_count + g) * N

            @pl.loop(0, ZROUNDS)
            def _(r):
                c = sid + r * NS

                @pl.when(c < NRC)
                def _():
                    pltpu.sync_copy(
                        acc.at[pl.ds(c * RB, RB), :],
                        out_hbm.at[pl.ds(obase + c * RB, RB), :])

            plsc.subcore_barrier()

    return k(x_flat, packed)


BN = 2000


def _tc_stage(partial, W, mask, relu):
    per_sample = partial.shape[1] == T

    def body(p_ref, w_ref, m_ref, o_ref):
        p = p_ref[0, 0] + p_ref[1, 0]
        pw = jnp.dot(p, w_ref[...] * m_ref[0],
                     preferred_element_type=jnp.float32)
        if relu:
            pw = jnp.maximum(pw, 0.0)
        o_ref[0] = pw

    if per_sample:
        p_map = lambda n, i: (0, i, n, 0)
    else:
        p_map = lambda n, i: (0, 0, n, 0)
    return pl.pallas_call(
        body,
        grid=(N // BN, T),
        in_specs=[
            pl.BlockSpec((2, 1, BN, D), p_map),
            pl.BlockSpec((D, D), lambda n, i: (0, 0)),
            pl.BlockSpec((1, D, D), lambda n, i: (i, 0, 0)),
        ],
        out_specs=pl.BlockSpec((1, BN, D), lambda n, i: (i, n, 0)),
        out_shape=jax.ShapeDtypeStruct((T, N, D), jnp.float32),
    )(partial, W, mask)


def kernel(feature, edge_weight, W1, W2, mask1, mask2, edge_index):
    w_bits = jax.lax.bitcast_convert_type(edge_weight, jnp.int32)
    packed = jnp.concatenate(
        [edge_index, w_bits[None]], axis=0)
    packed = packed.reshape(3, NW * NCHUNK, CH).transpose(1, 0, 2)
    packed = jnp.pad(packed.reshape(NW * NCHUNK, 3 * CH),
                     ((0, 0), (0, REC - 3 * CH))).reshape(-1)

    p1 = _sc_pass(feature, packed, 1)
    p1 = p1.reshape(NC, 1, N, D)

    H = _tc_stage(p1, W1, mask1, relu=True)

    p2 = _sc_pass(H.reshape(T * N, D), packed, T)
    p2 = p2.reshape(NC, T, N, D)

    return _tc_stage(p2, W2, mask2, relu=False)


# speedup vs baseline: 1.1548x; 1.0150x over previous
import dataclasses
import functools

import jax
import jax.numpy as jnp
from jax import lax
from jax.experimental import pallas as pl
from jax.experimental.pallas import tpu as pltpu
from jax.experimental.pallas import tpu_sc as plsc

N = 10000
E = 320000
D = 128
T = 4
NC = 2
NS = 16
NW = NC * NS
EPT = E // NW
CH = 80
NCHUNK = EPT // CH
RB = 80
NRC = N // RB
ZROUNDS = (NRC + NS - 1) // NS


NSLOT = 4
MAIN = NCHUNK - 1
NROUND = MAIN // NSLOT
REC = 256
ZB = 40
NZC = N // ZB
ZZROUNDS = (NZC + NS - 1) // NS


def _sc_pass(x_flat, packed, g_count):
    mesh = plsc.VectorSubcoreMesh(core_axis_name="c", subcore_axis_name="s")
    cp = pltpu.CompilerParams()
    if "needs_layout_passes" in pltpu.CompilerParams.__dataclass_fields__:
        cp = dataclasses.replace(cp, needs_layout_passes=False)

    @functools.partial(
        pl.kernel,
        mesh=mesh,
        compiler_params=cp,
        out_type=jax.ShapeDtypeStruct((NC * g_count * N, D), jnp.float32),
        scratch_types=(
            [pltpu.VMEM((REC,), jnp.int32)] * NSLOT
            + [pltpu.VMEM((CH,), jnp.int32)] * NSLOT
            + [pltpu.VMEM((CH,), jnp.int32)] * NSLOT
            + [pltpu.VMEM((CH, D), jnp.float32)] * NSLOT
            + [pltpu.VMEM((CH,), jnp.float32)] * NSLOT
            + [
                pltpu.VMEM((ZB, D), jnp.float32),
                pltpu.VMEM_SHARED((N, D), jnp.float32),
            ]
            + [pltpu.SemaphoreType.DMA] * (3 * NSLOT)
        ),
    )
    def k(x_hbm, pk_hbm, out_hbm, *scratch):
        idx_v = scratch[0:NSLOT]
        srcg = scratch[NSLOT:2 * NSLOT]
        dstb = scratch[2 * NSLOT:3 * NSLOT]
        rows_v = scratch[3 * NSLOT:4 * NSLOT]
        wbuf = scratch[4 * NSLOT:5 * NSLOT]
        zbuf_v = scratch[5 * NSLOT]
        acc = scratch[5 * NSLOT + 1]
        isem = scratch[5 * NSLOT + 2:5 * NSLOT + 2 + NSLOT]
        gsem = scratch[5 * NSLOT + 2 + NSLOT:5 * NSLOT + 2 + 2 * NSLOT]
        ssem = scratch[5 * NSLOT + 2 + 2 * NSLOT:]

        cid = lax.axis_index("c")
        sid = lax.axis_index("s")
        wid = cid * NS + sid
        cbase = wid * NCHUNK

        def idx_copy(c, s):
            off = (cbase + c) * REC
            return pltpu.make_async_copy(
                pk_hbm.at[pl.ds(off, REC)], idx_v[s], isem[s])

        def gather_copy(s):
            return pltpu.make_async_copy(
                x_hbm.at[srcg[s]], rows_v[s], gsem[s])

        def scatter_copy(s):
            return pltpu.make_async_copy(
                rows_v[s], acc.at[dstb[s]], ssem[s])

        def s1_prep(g, s):
            goff = jnp.full((16,), g * N, jnp.int32)
            for t in range(CH // 16):
                sl = pl.ds(t * 16, 16)
                srcg[s][sl] = idx_v[s][pl.ds(t * 16, 16)] + goff
                dstb[s][sl] = idx_v[s][pl.ds(CH + t * 16, 16)]
                wbuf[s][sl] = plsc.bitcast(
                    idx_v[s][pl.ds(2 * CH + t * 16, 16)], jnp.float32)
            gather_copy(s).start()

        def s2_scale(s):
            @plsc.parallel_loop(0, CH, 1, unroll=4)
            def _(j):
                wv = plsc.load_gather(
                    wbuf[s], [jnp.full((16,), j, jnp.int32)])
                for f in range(D // 16):
                    sl = (j, pl.ds(f * 16, 16))
                    rows_v[s][sl] = rows_v[s][sl] * wv

        @pl.loop(0, ZB)
        def _(r):
            for f in range(D // 16):
                zbuf_v[r, pl.ds(f * 16, 16)] = jnp.zeros((16,), jnp.float32)

        @pl.loop(0, g_count)
        def _(g):
            @pl.loop(0, ZZROUNDS)
            def _(r):
                c = sid + r * NS

                @pl.when(c < NZC)
                def _():
                    pltpu.sync_copy(zbuf_v, acc.at[pl.ds(c * ZB, ZB), :])

            plsc.subcore_barrier()

            idx_copy(0, 0).start()
            idx_copy(1, 1).start()
            idx_copy(2, 2).start()
            idx_copy(0, 0).wait()
            s1_prep(g, 0)
            idx_copy(1, 1).wait()
            s1_prep(g, 1)

            @pl.loop(0, NROUND)
            def _(t):
                c0 = t * NSLOT
                for p in range(NSLOT):
                    c = c0 + p
                    s = p
                    s2 = (p + 2) % NSLOT
                    s3 = (p + 3) % NSLOT

                    @pl.when(c + 3 < NCHUNK)
                    def _():
                        idx_copy(c + 3, s3).start()

                    @pl.when(c + 2 < NCHUNK)
                    def _():
                        idx_copy(c + 2, s2).wait()

                        @pl.when(c + 2 >= NSLOT)
                        def _():
                            scatter_copy(s2).wait()

                        s1_prep(g, s2)

                    gather_copy(s).wait()
                    s2_scale(s)
                    scatter_copy(s).start(add=True)

            gather_copy(0).wait()
            s2_scale(0)
            scatter_copy(0).start(add=True)

            for s in range(NSLOT):
                scatter_copy(s).wait()
            plsc.subcore_barrier()

            obase = (cid * g_count + g) * N

            @pl.loop(0, ZROUNDS)
            def _(r):
                c = sid + r * NS

                @pl.when(c < NRC)
                def _():
                    pltpu.sync_copy(
                        acc.at[pl.ds(c * RB, RB), :],
                        out_hbm.at[pl.ds(obase + c * RB, RB), :])

            plsc.subcore_barrier()

    return k(x_flat, packed)


BN = 5000


def _tc_stage(partial, W, mask, relu):
    per_sample = partial.shape[1] == T

    def body(p_ref, w_ref, m_ref, o_ref):
        p = p_ref[0, 0] + p_ref[1, 0]
        pw = jnp.dot(p, w_ref[...] * m_ref[0],
                     preferred_element_type=jnp.float32)
        if relu:
            pw = jnp.maximum(pw, 0.0)
        o_ref[0] = pw

    if per_sample:
        p_map = lambda n, i: (0, i, n, 0)
    else:
        p_map = lambda n, i: (0, 0, n, 0)
    return pl.pallas_call(
        body,
        grid=(N // BN, T),
        in_specs=[
            pl.BlockSpec((2, 1, BN, D), p_map),
            pl.BlockSpec((D, D), lambda n, i: (0, 0)),
            pl.BlockSpec((1, D, D), lambda n, i: (i, 0, 0)),
        ],
        out_specs=pl.BlockSpec((1, BN, D), lambda n, i: (i, n, 0)),
        out_shape=jax.ShapeDtypeStruct((T, N, D), jnp.float32),
    )(partial, W, mask)


def kernel(feature, edge_weight, W1, W2, mask1, mask2, edge_index):
    w_bits = jax.lax.bitcast_convert_type(edge_weight, jnp.int32)
    packed = jnp.concatenate(
        [edge_index, w_bits[None]], axis=0)
    packed = packed.reshape(3, NW * NCHUNK, CH).transpose(1, 0, 2)
    packed = jnp.pad(packed.reshape(NW * NCHUNK, 3 * CH),
                     ((0, 0), (0, REC - 3 * CH))).reshape(-1)

    p1 = _sc_pass(feature, packed, 1)
    p1 = p1.reshape(NC, 1, N, D)

    H = _tc_stage(p1, W1, mask1, relu=True)

    p2 = _sc_pass(H.reshape(T * N, D), packed, T)
    p2 = p2.reshape(NC, T, N, D)

    return _tc_stage(p2, W2, mask2, relu=False)


# speedup vs baseline: 1.1646x; 1.0085x over previous
import dataclasses
import functools

import jax
import jax.numpy as jnp
from jax import lax
from jax.experimental import pallas as pl
from jax.experimental.pallas import tpu as pltpu
from jax.experimental.pallas import tpu_sc as plsc

N = 10000
E = 320000
D = 128
T = 4
NC = 2
NS = 16
NW = NC * NS
EPT = E // NW
CH = 80
NCHUNK = EPT // CH
RB = 80
NRC = N // RB
ZROUNDS = (NRC + NS - 1) // NS


NSLOT = 4
MAIN = NCHUNK - 1
NROUND = MAIN // NSLOT
REC = 256
ZB = 40
NZC = N // ZB
ZZROUNDS = (NZC + NS - 1) // NS


def _sc_pass(x_flat, packed, g_count):
    mesh = plsc.VectorSubcoreMesh(core_axis_name="c", subcore_axis_name="s")
    cp = pltpu.CompilerParams()
    if "needs_layout_passes" in pltpu.CompilerParams.__dataclass_fields__:
        cp = dataclasses.replace(cp, needs_layout_passes=False)

    @functools.partial(
        pl.kernel,
        mesh=mesh,
        compiler_params=cp,
        out_type=jax.ShapeDtypeStruct((NC * g_count * N, D), jnp.float32),
        scratch_types=(
            [pltpu.VMEM((REC,), jnp.int32)] * NSLOT
            + [pltpu.VMEM((CH,), jnp.int32)] * NSLOT
            + [pltpu.VMEM((CH,), jnp.int32)] * NSLOT
            + [pltpu.VMEM((CH, D), jnp.float32)] * NSLOT
            + [pltpu.VMEM((CH,), jnp.float32)] * NSLOT
            + [
                pltpu.VMEM((ZB, D), jnp.float32),
                pltpu.VMEM_SHARED((N, D), jnp.float32),
            ]
            + [pltpu.SemaphoreType.DMA] * (3 * NSLOT)
        ),
    )
    def k(x_hbm, pk_hbm, out_hbm, *scratch):
        idx_v = scratch[0:NSLOT]
        srcg = scratch[NSLOT:2 * NSLOT]
        dstb = scratch[2 * NSLOT:3 * NSLOT]
        rows_v = scratch[3 * NSLOT:4 * NSLOT]
        wbuf = scratch[4 * NSLOT:5 * NSLOT]
        zbuf_v = scratch[5 * NSLOT]
        acc = scratch[5 * NSLOT + 1]
        isem = scratch[5 * NSLOT + 2:5 * NSLOT + 2 + NSLOT]
        gsem = scratch[5 * NSLOT + 2 + NSLOT:5 * NSLOT + 2 + 2 * NSLOT]
        ssem = scratch[5 * NSLOT + 2 + 2 * NSLOT:]

        cid = lax.axis_index("c")
        sid = lax.axis_index("s")
        wid = cid * NS + sid
        cbase = wid * NCHUNK

        def idx_copy(c, s):
            off = (cbase + c) * REC
            return pltpu.make_async_copy(
                pk_hbm.at[pl.ds(off, REC)], idx_v[s], isem[s])

        def gather_copy(s):
            return pltpu.make_async_copy(
                x_hbm.at[srcg[s]], rows_v[s], gsem[s])

        def scatter_copy(s):
            return pltpu.make_async_copy(
                rows_v[s], acc.at[dstb[s]], ssem[s])

        def s1_prep(g, s):
            goff = jnp.full((16,), g * N, jnp.int32)
            for t in range(CH // 16):
                sl = pl.ds(t * 16, 16)
                srcg[s][sl] = idx_v[s][pl.ds(t * 16, 16)] + goff
                dstb[s][sl] = idx_v[s][pl.ds(CH + t * 16, 16)]
                wbuf[s][sl] = plsc.bitcast(
                    idx_v[s][pl.ds(2 * CH + t * 16, 16)], jnp.float32)
            gather_copy(s).start()

        def s2_scale(s):
            @plsc.parallel_loop(0, CH, 1, unroll=4)
            def _(j):
                wv = plsc.load_gather(
                    wbuf[s], [jnp.full((16,), j, jnp.int32)])
                for f in range(D // 16):
                    sl = (j, pl.ds(f * 16, 16))
                    rows_v[s][sl] = rows_v[s][sl] * wv

        @pl.loop(0, ZB)
        def _(r):
            for f in range(D // 16):
                zbuf_v[r, pl.ds(f * 16, 16)] = jnp.zeros((16,), jnp.float32)

        @pl.loop(0, g_count)
        def _(g):
            @pl.loop(0, ZZROUNDS)
            def _(r):
                c = sid + r * NS

                @pl.when(c < NZC)
                def _():
                    pltpu.sync_copy(zbuf_v, acc.at[pl.ds(c * ZB, ZB), :])

            plsc.subcore_barrier()

            idx_copy(0, 0).start()
            idx_copy(1, 1).start()
            idx_copy(2, 2).start()
            idx_copy(0, 0).wait()
            s1_prep(g, 0)
            idx_copy(1, 1).wait()
            s1_prep(g, 1)

            @pl.loop(0, NROUND)
            def _(t):
                c0 = t * NSLOT
                for p in range(NSLOT):
                    c = c0 + p
                    s = p
                    s2 = (p + 2) % NSLOT
                    s3 = (p + 3) % NSLOT

                    @pl.when(c + 3 < NCHUNK)
                    def _():
                        idx_copy(c + 3, s3).start()

                    @pl.when(c + 2 < NCHUNK)
                    def _():
                        idx_copy(c + 2, s2).wait()

                        @pl.when(c + 2 >= NSLOT)
                        def _():
                            scatter_copy(s2).wait()

                        s1_prep(g, s2)

                    gather_copy(s).wait()
                    s2_scale(s)
                    scatter_copy(s).start(add=True)

            gather_copy(0).wait()
            s2_scale(0)
            scatter_copy(0).start(add=True)

            for s in range(NSLOT):
                scatter_copy(s).wait()
            plsc.subcore_barrier()

            obase = (cid * g_count + g) * N

            @pl.loop(0, ZROUNDS)
            def _(r):
                c = sid + r * NS

                @pl.when(c < NRC)
                def _():
                    pltpu.sync_copy(
                        acc.at[pl.ds(c * RB, RB), :],
                        out_hbm.at[pl.ds(obase + c * RB, RB), :])

            plsc.subcore_barrier()

    return k(x_flat, packed)


BN = 10000


def _tc_stage(partial, W, mask, relu):
    per_sample = partial.shape[1] == T

    def body(p_ref, w_ref, m_ref, o_ref):
        p = p_ref[0, 0] + p_ref[1, 0]
        pw = jnp.dot(p, w_ref[...] * m_ref[0],
                     preferred_element_type=jnp.float32)
        if relu:
            pw = jnp.maximum(pw, 0.0)
        o_ref[0] = pw

    if per_sample:
        p_map = lambda n, i: (0, i, n, 0)
    else:
        p_map = lambda n, i: (0, 0, n, 0)
    return pl.pallas_call(
        body,
        grid=(N // BN, T),
        in_specs=[
            pl.BlockSpec((2, 1, BN, D), p_map),
            pl.BlockSpec((D, D), lambda n, i: (0, 0)),
            pl.BlockSpec((1, D, D), lambda n, i: (i, 0, 0)),
        ],
        out_specs=pl.BlockSpec((1, BN, D), lambda n, i: (i, n, 0)),
        out_shape=jax.ShapeDtypeStruct((T, N, D), jnp.float32),
    )(partial, W, mask)


def kernel(feature, edge_weight, W1, W2, mask1, mask2, edge_index):
    w_bits = jax.lax.bitcast_convert_type(edge_weight, jnp.int32)
    packed = jnp.concatenate(
        [edge_index, w_bits[None]], axis=0)
    packed = packed.reshape(3, NW * NCHUNK, CH).transpose(1, 0, 2)
    packed = jnp.pad(packed.reshape(NW * NCHUNK, 3 * CH),
                     ((0, 0), (0, REC - 3 * CH))).reshape(-1)

    p1 = _sc_pass(feature, packed, 1)
    p1 = p1.reshape(NC, 1, N, D)

    H = _tc_stage(p1, W1, mask1, relu=True)

    p2 = _sc_pass(H.reshape(T * N, D), packed, T)
    p2 = p2.reshape(NC, T, N, D)

    return _tc_stage(p2, W2, mask2, relu=False)


# speedup vs baseline: 1.1797x; 1.0130x over previous
import dataclasses
import functools

import jax
import jax.numpy as jnp
from jax import lax
from jax.experimental import pallas as pl
from jax.experimental.pallas import tpu as pltpu
from jax.experimental.pallas import tpu_sc as plsc

N = 10000
E = 320000
D = 128
T = 4
NC = 2
NS = 16
NW = NC * NS
EPT = E // NW
CH = 80
NCHUNK = EPT // CH
RB = 80
NRC = N // RB
ZROUNDS = (NRC + NS - 1) // NS


NSLOT = 4
MAIN = NCHUNK - 1
NROUND = MAIN // NSLOT
REC = 256
ZB = 40
NZC = N // ZB
ZZROUNDS = (NZC + NS - 1) // NS


def _sc_pass(x_flat, packed, g_count):
    mesh = plsc.VectorSubcoreMesh(core_axis_name="c", subcore_axis_name="s")
    cp = pltpu.CompilerParams()
    if "needs_layout_passes" in pltpu.CompilerParams.__dataclass_fields__:
        cp = dataclasses.replace(cp, needs_layout_passes=False)

    @functools.partial(
        pl.kernel,
        mesh=mesh,
        compiler_params=cp,
        out_type=jax.ShapeDtypeStruct((NC * g_count * N, D), jnp.float32),
        scratch_types=(
            [pltpu.VMEM((REC,), jnp.int32)] * NSLOT
            + [pltpu.VMEM((CH,), jnp.int32)] * NSLOT
            + [pltpu.VMEM((CH,), jnp.int32)] * NSLOT
            + [pltpu.VMEM((CH, D), jnp.float32)] * NSLOT
            + [pltpu.VMEM((CH,), jnp.float32)] * NSLOT
            + [
                pltpu.VMEM((ZB, D), jnp.float32),
                pltpu.VMEM_SHARED((N, D), jnp.float32),
            ]
            + [pltpu.SemaphoreType.DMA] * (3 * NSLOT)
        ),
    )
    def k(x_hbm, pk_hbm, out_hbm, *scratch):
        idx_v = scratch[0:NSLOT]
        srcg = scratch[NSLOT:2 * NSLOT]
        dstb = scratch[2 * NSLOT:3 * NSLOT]
        rows_v = scratch[3 * NSLOT:4 * NSLOT]
        wbuf = scratch[4 * NSLOT:5 * NSLOT]
        zbuf_v = scratch[5 * NSLOT]
        acc = scratch[5 * NSLOT + 1]
        isem = scratch[5 * NSLOT + 2:5 * NSLOT + 2 + NSLOT]
        gsem = scratch[5 * NSLOT + 2 + NSLOT:5 * NSLOT + 2 + 2 * NSLOT]
        ssem = scratch[5 * NSLOT + 2 + 2 * NSLOT:]

        cid = lax.axis_index("c")
        sid = lax.axis_index("s")
        wid = cid * NS + sid
        cbase = wid * NCHUNK

        def idx_copy(c, s):
            off = (cbase + c) * REC
            return pltpu.make_async_copy(
                pk_hbm.at[pl.ds(off, REC)], idx_v[s], isem[s])

        def gather_copy(s):
            return pltpu.make_async_copy(
                x_hbm.at[srcg[s]], rows_v[s], gsem[s])

        def scatter_copy(s):
            return pltpu.make_async_copy(
                rows_v[s], acc.at[dstb[s]], ssem[s])

        def s1_prep(g, s):
            goff = jnp.full((16,), g * N, jnp.int32)
            for t in range(CH // 16):
                sl = pl.ds(t * 16, 16)
                srcg[s][sl] = idx_v[s][pl.ds(t * 16, 16)] + goff
                dstb[s][sl] = idx_v[s][pl.ds(CH + t * 16, 16)]
                wbuf[s][sl] = plsc.bitcast(
                    idx_v[s][pl.ds(2 * CH + t * 16, 16)], jnp.float32)
            gather_copy(s).start()

        def s2_scale(s):
            @plsc.parallel_loop(0, CH, 1, unroll=4)
            def _(j):
                wv = plsc.load_gather(
                    wbuf[s], [jnp.full((16,), j, jnp.int32)])
                for f in range(D // 16):
                    sl = (j, pl.ds(f * 16, 16))
                    rows_v[s][sl] = rows_v[s][sl] * wv

        @pl.loop(0, ZB)
        def _(r):
            for f in range(D // 16):
                zbuf_v[r, pl.ds(f * 16, 16)] = jnp.zeros((16,), jnp.float32)

        @pl.loop(0, g_count)
        def _(g):
            idx_copy(0, 0).start()
            idx_copy(1, 1).start()
            idx_copy(2, 2).start()
            idx_copy(0, 0).wait()
            s1_prep(g, 0)
            idx_copy(1, 1).wait()
            s1_prep(g, 1)

            @pl.loop(0, ZZROUNDS)
            def _(r):
                c = sid + r * NS

                @pl.when(c < NZC)
                def _():
                    pltpu.sync_copy(zbuf_v, acc.at[pl.ds(c * ZB, ZB), :])

            plsc.subcore_barrier()

            @pl.loop(0, NROUND)
            def _(t):
                c0 = t * NSLOT
                for p in range(NSLOT):
                    c = c0 + p
                    s = p
                    s2 = (p + 2) % NSLOT
                    s3 = (p + 3) % NSLOT

                    @pl.when(c + 3 < NCHUNK)
                    def _():
                        idx_copy(c + 3, s3).start()

                    @pl.when(c + 2 < NCHUNK)
                    def _():
                        idx_copy(c + 2, s2).wait()

                        @pl.when(c + 2 >= NSLOT)
                        def _():
                            scatter_copy(s2).wait()

                        s1_prep(g, s2)

                    gather_copy(s).wait()
                    s2_scale(s)
                    scatter_copy(s).start(add=True)

            gather_copy(0).wait()
            s2_scale(0)
            scatter_copy(0).start(add=True)

            for s in range(NSLOT):
                scatter_copy(s).wait()
            plsc.subcore_barrier()

            obase = (cid * g_count + g) * N

            @pl.loop(0, ZROUNDS)
            def _(r):
                c = sid + r * NS

                @pl.when(c < NRC)
                def _():
                    pltpu.sync_copy(
                        acc.at[pl.ds(c * RB, RB), :],
                        out_hbm.at[pl.ds(obase + c * RB, RB), :])

            plsc.subcore_barrier()

    return k(x_flat, packed)


BN = 10000


def _tc_stage(partial, W, mask, relu):
    per_sample = partial.shape[1] == T

    def body(p_ref, w_ref, m_ref, o_ref):
        p = p_ref[0, 0] + p_ref[1, 0]
        pw = jnp.dot(p, w_ref[...] * m_ref[0],
                     preferred_element_type=jnp.float32)
        if relu:
            pw = jnp.maximum(pw, 0.0)
        o_ref[0] = pw

    if per_sample:
        p_map = lambda n, i: (0, i, n, 0)
    else:
        p_map = lambda n, i: (0, 0, n, 0)
    return pl.pallas_call(
        body,
        grid=(N // BN, T),
        in_specs=[
            pl.BlockSpec((2, 1, BN, D), p_map),
            pl.BlockSpec((D, D), lambda n, i: (0, 0)),
            pl.BlockSpec((1, D, D), lambda n, i: (i, 0, 0)),
        ],
        out_specs=pl.BlockSpec((1, BN, D), lambda n, i: (i, n, 0)),
        out_shape=jax.ShapeDtypeStruct((T, N, D), jnp.float32),
    )(partial, W, mask)


def kernel(feature, edge_weight, W1, W2, mask1, mask2, edge_index):
    w_bits = jax.lax.bitcast_convert_type(edge_weight, jnp.int32)
    packed = jnp.concatenate(
        [edge_index, w_bits[None]], axis=0)
    packed = packed.reshape(3, NW * NCHUNK, CH).transpose(1, 0, 2)
    packed = jnp.pad(packed.reshape(NW * NCHUNK, 3 * CH),
                     ((0, 0), (0, REC - 3 * CH))).reshape(-1)

    p1 = _sc_pass(feature, packed, 1)
    p1 = p1.reshape(NC, 1, N, D)

    H = _tc_stage(p1, W1, mask1, relu=True)

    p2 = _sc_pass(H.reshape(T * N, D), packed, T)
    p2 = p2.reshape(NC, T, N, D)

    return _tc_stage(p2, W2, mask2, relu=False)
